# baseline (speedup 1.0000x reference)
import functools

import jax
import jax.numpy as jnp
from jax import lax
from jax.experimental import pallas as pl
from jax.experimental.pallas import tpu as pltpu
from jax.experimental.pallas import tpu_sc as plsc

BATCH = 16384
N_FIELDS = 26
EMBED_DIM = 32
TOTAL = BATCH * N_FIELDS
NC, NS = 2, 16
NW = NC * NS
PER_W = TOTAL // NW
CHUNK = 128
N_CHUNKS = PER_W // CHUNK
VECS = CHUNK // 16
PERIOD_V = 13


def _sc_gather(idx2d, off_pat, table):
    mesh = plsc.VectorSubcoreMesh(core_axis_name="c", subcore_axis_name="s")

    @functools.partial(
        pl.kernel,
        out_type=jax.ShapeDtypeStruct((TOTAL, EMBED_DIM), jnp.float32),
        mesh=mesh,
        compiler_params=pltpu.CompilerParams(use_tc_tiling_on_sc=False),
        scratch_types=[
            pltpu.VMEM((N_CHUNKS, CHUNK), jnp.int32),
            pltpu.VMEM((PERIOD_V, 16), jnp.int32),
            pltpu.VMEM((CHUNK, EMBED_DIM), jnp.float32),
            pltpu.SemaphoreType.DMA,
        ],
    )
    def k(idx_hbm, offs_hbm, table_hbm, out_hbm, idx_v, offs_v, rows_v, sem):
        c = lax.axis_index("c")
        s = lax.axis_index("s")
        wid = s * NC + c
        base = wid * PER_W
        pltpu.sync_copy(idx_hbm.at[pl.ds(wid * N_CHUNKS, N_CHUNKS)], idx_v)
        pltpu.sync_copy(offs_hbm, offs_v)

        def chunk_body(j, _):
            for u in range(VECS):
                r = lax.rem(j * VECS + u, PERIOD_V)
                off = offs_v[r, :]
                idx_v[j, pl.ds(u * 16, 16)] = idx_v[j, pl.ds(u * 16, 16)] + off
            pltpu.async_copy(table_hbm.at[idx_v.at[j]], rows_v, sem).wait()
            pltpu.sync_copy(rows_v, out_hbm.at[pl.ds(base + j * CHUNK, CHUNK)])
            return 0

        lax.fori_loop(0, N_CHUNKS, chunk_body, 0)

    return k(idx2d, off_pat, table)


def kernel(sparse_indices, offsets, embed_table):
    idx2d = sparse_indices.reshape(TOTAL // CHUNK, CHUNK)
    off_pat = jnp.tile(offsets.reshape(N_FIELDS), PERIOD_V * 16 // N_FIELDS)
    off_pat = off_pat.reshape(PERIOD_V, 16)
    out = _sc_gather(idx2d, off_pat, embed_table)
    return out.reshape(BATCH, N_FIELDS, EMBED_DIM)


# speedup vs baseline: 1.0531x; 1.0531x over previous
import functools

import jax
import jax.numpy as jnp
from jax import lax
from jax.experimental import pallas as pl
from jax.experimental.pallas import tpu as pltpu
from jax.experimental.pallas import tpu_sc as plsc

BATCH = 16384
N_FIELDS = 26
EMBED_DIM = 32
TOTAL = BATCH * N_FIELDS
NC, NS = 2, 16
NW = NC * NS
PER_W = TOTAL // NW
CHUNK = 128
N_CHUNKS = PER_W // CHUNK
VECS = CHUNK // 16
PERIOD_V = 13
NBUF = 8
ROUNDS = N_CHUNKS // NBUF


def _sc_gather(idx2d, off_pat, table):
    mesh = plsc.VectorSubcoreMesh(core_axis_name="c", subcore_axis_name="s")

    @functools.partial(
        pl.kernel,
        out_type=jax.ShapeDtypeStruct((TOTAL, EMBED_DIM), jnp.float32),
        mesh=mesh,
        compiler_params=pltpu.CompilerParams(use_tc_tiling_on_sc=False),
        scratch_types=[
            pltpu.VMEM((N_CHUNKS, CHUNK), jnp.int32),
            pltpu.VMEM((PERIOD_V, 16), jnp.int32),
            pltpu.VMEM((NBUF, CHUNK, EMBED_DIM), jnp.float32),
            pltpu.SemaphoreType.DMA((NBUF,)),
            pltpu.SemaphoreType.DMA((NBUF,)),
        ],
    )
    def k(idx_hbm, offs_hbm, table_hbm, out_hbm, idx_v, offs_v, rows_v,
          gsem, wsem):
        c = lax.axis_index("c")
        s = lax.axis_index("s")
        wid = s * NC + c
        base = wid * PER_W
        pltpu.sync_copy(idx_hbm.at[pl.ds(wid * N_CHUNKS, N_CHUNKS)], idx_v)
        pltpu.sync_copy(offs_hbm, offs_v)

        def add_offsets(j):
            for u in range(VECS):
                rp = lax.rem(j * VECS + u, PERIOD_V)
                idx_v[j, pl.ds(u * 16, 16)] = (
                    idx_v[j, pl.ds(u * 16, 16)] + offs_v[rp, :]
                )

        def gather_desc(j, b):
            return pltpu.make_async_copy(
                table_hbm.at[idx_v.at[j]], rows_v.at[b], gsem.at[b]
            )

        def write_desc(j, b):
            return pltpu.make_async_copy(
                rows_v.at[b],
                out_hbm.at[pl.ds(base + j * CHUNK, CHUNK)],
                wsem.at[b],
            )

        for b in range(NBUF):
            add_offsets(b)
        for b in range(NBUF):
            gather_desc(b, b).start()

        def round_body(r, _):
            @pl.when(r < ROUNDS - 1)
            def _offs():
                for b in range(NBUF):
                    add_offsets((r + 1) * NBUF + b)

            for b in range(NBUF):
                j = r * NBUF + b
                gather_desc(j, b).wait()
                write_desc(j, b).start()
            for b in range(NBUF):
                j = r * NBUF + b
                write_desc(j, b).wait()

                @pl.when(r < ROUNDS - 1)
                def _refill():
                    gather_desc((r + 1) * NBUF + b, b).start()

            return 0

        lax.fori_loop(0, ROUNDS, round_body, 0)

    return k(idx2d, off_pat, table)


def kernel(sparse_indices, offsets, embed_table):
    idx2d = sparse_indices.reshape(TOTAL // CHUNK, CHUNK)
    off_pat = jnp.tile(offsets.reshape(N_FIELDS), PERIOD_V * 16 // N_FIELDS)
    off_pat = off_pat.reshape(PERIOD_V, 16)
    out = _sc_gather(idx2d, off_pat, embed_table)
    return out.reshape(BATCH, N_FIELDS, EMBED_DIM)
